# baseline (speedup 1.0000x reference)
import functools

import jax
import jax.numpy as jnp
from jax import lax
from jax.experimental import pallas as pl
from jax.experimental.pallas import tpu as pltpu
from jax.experimental.pallas import tpu_sc as plsc

B, S, D = 1, 2048, 768
NH, DH = 12, 64
M = 32768
K = 32
DFF = 3072

CH = 128
NCHUNK = M // CH

NEG_INF = float(jnp.finfo(jnp.float32).min)
BIG_F = float(jnp.finfo(jnp.float32).max)

NC, NS = 2, 16
NW = NC * NS


def _ln_qkv_body(x_ref, g_ref, b_ref, w_ref, wb_ref, qkv_ref):
    x = x_ref[...]
    mu = jnp.mean(x, axis=-1, keepdims=True)
    var = jnp.mean((x - mu) ** 2, axis=-1, keepdims=True)
    h = (x - mu) * jax.lax.rsqrt(var + 1e-5) * g_ref[...] + b_ref[...]
    qkv_ref[...] = (
        jnp.dot(h, w_ref[...], preferred_element_type=jnp.float32) + wb_ref[...]
    )


def _ln_qkv(x, g, b, w, wb):
    blk = 256
    return pl.pallas_call(
        _ln_qkv_body,
        grid=(S // blk,),
        in_specs=[
            pl.BlockSpec((blk, D), lambda i: (i, 0)),
            pl.BlockSpec((1, D), lambda i: (0, 0)),
            pl.BlockSpec((1, D), lambda i: (0, 0)),
            pl.BlockSpec((D, 3 * D), lambda i: (0, 0)),
            pl.BlockSpec((1, 3 * D), lambda i: (0, 0)),
        ],
        out_specs=pl.BlockSpec((blk, 3 * D), lambda i: (i, 0)),
        out_shape=jax.ShapeDtypeStruct((S, 3 * D), jnp.float32),
    )(x, g.reshape(1, D), b.reshape(1, D), w, wb.reshape(1, 3 * D))


def _scores_body(q_ref, k_ref, s_ref, cm_ref, *, sblk, mblk):
    q = q_ref[...]
    k = k_ref[:, 0, :]
    s = jax.lax.dot_general(
        q, k, (((1,), (1,)), ((), ())), preferred_element_type=jnp.float32
    )
    s_ref[...] = s
    cm = jnp.max(s.reshape(sblk, mblk // CH, CH), axis=-1)
    cm_ref[...] = cm.T


def _scores(q, db_kv):
    sblk, mblk = 256, 2048
    return pl.pallas_call(
        functools.partial(_scores_body, sblk=sblk, mblk=mblk),
        grid=(M // mblk, S // sblk),
        in_specs=[
            pl.BlockSpec((sblk, D), lambda m, s: (s, 0)),
            pl.BlockSpec((mblk, 2, D), lambda m, s: (m, 0, 0)),
        ],
        out_specs=[
            pl.BlockSpec((sblk, mblk), lambda m, s: (s, m)),
            pl.BlockSpec((mblk // CH, sblk), lambda m, s: (m, s)),
        ],
        out_shape=[
            jax.ShapeDtypeStruct((S, M), jnp.float32),
            jax.ShapeDtypeStruct((NCHUNK, S), jnp.float32),
        ],
    )(q, db_kv)


def _t1_body(cm_ref, o_ref):
    cm = cm_ref[...]
    iota = jax.lax.broadcasted_iota(jnp.int32, (NCHUNK, S), 0)
    lane = jax.lax.broadcasted_iota(jnp.int32, (1, S), 1)
    big = jnp.int32(2**30)
    rows = []
    for _ in range(K):
        m = jnp.max(cm, axis=0, keepdims=True)
        pos = jnp.where(cm == m, iota, big)
        amin = jnp.min(pos, axis=0, keepdims=True)
        rows.append(amin + lane * NCHUNK)
        cm = jnp.where(iota == amin, NEG_INF, cm)
    o_ref[...] = jnp.concatenate(rows, axis=0)


def _topchunks(cm_t):
    return pl.pallas_call(
        _t1_body,
        grid=(1,),
        in_specs=[pl.BlockSpec((NCHUNK, S), lambda i: (0, 0))],
        out_specs=pl.BlockSpec((K, S), lambda i: (0, 0)),
        out_shape=jax.ShapeDtypeStruct((K, S), jnp.int32),
    )(cm_t)


def _sc_gather(table, idx, chunk):
    row_shape = table.shape[1:]
    N = idx.shape[0]
    n_per_w = N // NW
    nch = n_per_w // chunk
    mesh = plsc.VectorSubcoreMesh(
        core_axis_name="c", subcore_axis_name="s", num_cores=NC,
        num_subcores=NS)

    npair = nch // 2

    @functools.partial(
        pl.kernel, mesh=mesh,
        out_type=jax.ShapeDtypeStruct((N,) + row_shape, jnp.float32),
        scratch_types=[
            pltpu.VMEM((chunk,), jnp.int32),
            pltpu.VMEM((chunk,), jnp.int32),
            pltpu.VMEM((chunk,) + row_shape, jnp.float32),
            pltpu.VMEM((chunk,) + row_shape, jnp.float32),
            pltpu.SemaphoreType.DMA,
            pltpu.SemaphoreType.DMA,
        ],
    )
    def k(table_hbm, idx_hbm, out_hbm, ic0, ic1, b0, b1, sem0, sem1):
        wid = lax.axis_index("s") * NC + lax.axis_index("c")
        base = wid * n_per_w

        def start(icr, bufr, semr, c):
            pltpu.sync_copy(idx_hbm.at[pl.ds(base + c * chunk, chunk)], icr)
            pltpu.async_copy(table_hbm.at[icr], bufr, semr)

        def drain(icr, bufr, semr, c):
            pltpu.make_async_copy(table_hbm.at[icr], bufr, semr).wait()
            pltpu.sync_copy(bufr, out_hbm.at[pl.ds(base + c * chunk, chunk)])

        start(ic0, b0, sem0, 0)

        def body(p, carry):
            start(ic1, b1, sem1, 2 * p + 1)
            drain(ic0, b0, sem0, 2 * p)

            @pl.when(p < npair - 1)
            def _():
                start(ic0, b0, sem0, 2 * p + 2)

            drain(ic1, b1, sem1, 2 * p + 1)
            return carry

        lax.fori_loop(0, npair, body, 0)

    return k(table, idx)


def _t2_body(cand_ref, flat_ref, o_ref, *, rblk):
    qb = pl.program_id(0)
    cand = cand_ref[...]
    lane = jax.lax.broadcasted_iota(jnp.int32, (1, rblk, CH), 2)
    l_io = jax.lax.broadcasted_iota(jnp.int32, (K, rblk, 1), 1) + qb * rblk
    cid = flat_ref[...].reshape(K, rblk, 1) - l_io * NCHUNK
    g3 = cid * CH + lane
    cols = []
    for _ in range(K):
        m1 = jnp.max(cand, axis=2, keepdims=True)
        m = jnp.max(m1, axis=0, keepdims=True)
        hit = cand == m
        s1 = jnp.min(jnp.where(hit, g3, jnp.int32(2**30)), axis=2,
                     keepdims=True)
        sel = jnp.min(s1, axis=0, keepdims=True)
        cols.append(sel.reshape(rblk, 1))
        cand = jnp.where(g3 == sel, NEG_INF, cand)
    o_ref[...] = jnp.concatenate(cols, axis=-1)


def _topcand(cand3, flat):
    rblk = 256
    return pl.pallas_call(
        functools.partial(_t2_body, rblk=rblk),
        grid=(S // rblk,),
        in_specs=[
            pl.BlockSpec((K, rblk, CH), lambda i: (0, i, 0)),
            pl.BlockSpec((K, rblk), lambda i: (0, i)),
        ],
        out_specs=pl.BlockSpec((rblk, K), lambda i: (i, 0)),
        out_shape=jax.ShapeDtypeStruct((S, K), jnp.int32),
    )(cand3, flat)


def _memattn_body(q_ref, kv_ref, o_ref):
    q = q_ref[...]
    outs = []
    scale = 1.0 / jnp.sqrt(jnp.float32(DH))
    for h in range(NH):
        qh = q[:, h * DH:(h + 1) * DH]
        mkh = kv_ref[:, :, 0, h * DH:(h + 1) * DH]
        mvh = kv_ref[:, :, 1, h * DH:(h + 1) * DH]
        aw = jnp.sum(qh[:, None, :] * mkh, axis=-1) * scale
        aw = aw - jnp.max(aw, axis=-1, keepdims=True)
        aw = jnp.exp(aw)
        aw = aw / jnp.sum(aw, axis=-1, keepdims=True)
        outs.append(jnp.sum(aw[:, :, None] * mvh, axis=1))
    o_ref[...] = jnp.concatenate(outs, axis=-1)


def _memattn(q, mem_kv_flat):
    blk = 64
    return pl.pallas_call(
        _memattn_body,
        grid=(S // blk,),
        in_specs=[
            pl.BlockSpec((blk, D), lambda i: (i, 0)),
            pl.BlockSpec((blk, K, 2, D), lambda i: (i, 0, 0, 0)),
        ],
        out_specs=pl.BlockSpec((blk, D), lambda i: (i, 0)),
        out_shape=jax.ShapeDtypeStruct((S, D), jnp.float32),
    )(q, mem_kv_flat)


def _causal_body(q_ref, k_ref, v_ref, am_ref, hm_ref, o_ref, *, qblk):
    qb = pl.program_id(0)
    rows = jax.lax.broadcasted_iota(jnp.int32, (qblk, S), 0) + qb * qblk
    cols = jax.lax.broadcasted_iota(jnp.int32, (qblk, S), 1)
    causal = rows >= cols
    am = am_ref[...]
    scale = 1.0 / jnp.sqrt(jnp.float32(DH))
    outs = []
    for h in range(NH):
        qh = q_ref[:, h * DH:(h + 1) * DH]
        kh = k_ref[:, h * DH:(h + 1) * DH]
        vh = v_ref[:, h * DH:(h + 1) * DH]
        logits = jax.lax.dot_general(
            qh, kh, (((1,), (1,)), ((), ())), preferred_element_type=jnp.float32
        ) * scale
        logits = jnp.where(causal, logits, NEG_INF) + am
        m = jnp.max(logits, axis=-1, keepdims=True)
        p = jnp.exp(logits - m)
        p = p / jnp.sum(p, axis=-1, keepdims=True)
        p = p * hm_ref[0, h]
        outs.append(jnp.dot(p, vh, preferred_element_type=jnp.float32))
    o_ref[...] = jnp.concatenate(outs, axis=-1)


def _causal_attn(q, k, v, amask, hmask):
    qblk = 256
    return pl.pallas_call(
        functools.partial(_causal_body, qblk=qblk),
        grid=(S // qblk,),
        in_specs=[
            pl.BlockSpec((qblk, D), lambda i: (i, 0)),
            pl.BlockSpec((S, D), lambda i: (0, 0)),
            pl.BlockSpec((S, D), lambda i: (0, 0)),
            pl.BlockSpec((1, S), lambda i: (0, 0)),
            pl.BlockSpec((1, NH), lambda i: (0, 0)),
        ],
        out_specs=pl.BlockSpec((qblk, D), lambda i: (i, 0)),
        out_shape=jax.ShapeDtypeStruct((S, D), jnp.float32),
    )(q, k, v, amask.reshape(1, S), hmask.reshape(1, NH))


def _tail_body(stdh_ref, mem_ref, res_ref, pw_ref, pb_ref, g_ref,
               g2_ref, b2_ref, w1_ref, b1_ref, w2_ref, bb2_ref, o_ref):
    std = (
        jnp.dot(stdh_ref[...], pw_ref[...], preferred_element_type=jnp.float32)
        + pb_ref[...]
    )
    g = g_ref[0, 0]
    attn = (1.0 - g) * std + g * mem_ref[...]
    hidden = attn + res_ref[...]
    mu = jnp.mean(hidden, axis=-1, keepdims=True)
    var = jnp.mean((hidden - mu) ** 2, axis=-1, keepdims=True)
    h2 = (hidden - mu) * jax.lax.rsqrt(var + 1e-5) * g2_ref[...] + b2_ref[...]
    ff = jnp.dot(h2, w1_ref[...], preferred_element_type=jnp.float32) + b1_ref[...]
    ff = jax.nn.gelu(ff, approximate=True)
    ff = jnp.dot(ff, w2_ref[...], preferred_element_type=jnp.float32) + bb2_ref[...]
    o_ref[...] = hidden + ff


def _tail(stdh, mem, res, pw, pb, g_val, g2, b2, w1, b1, w2, bb2):
    blk = 256
    return pl.pallas_call(
        _tail_body,
        grid=(S // blk,),
        in_specs=[
            pl.BlockSpec((blk, D), lambda i: (i, 0)),
            pl.BlockSpec((blk, D), lambda i: (i, 0)),
            pl.BlockSpec((blk, D), lambda i: (i, 0)),
            pl.BlockSpec((D, D), lambda i: (0, 0)),
            pl.BlockSpec((1, D), lambda i: (0, 0)),
            pl.BlockSpec((1, 1), lambda i: (0, 0)),
            pl.BlockSpec((1, D), lambda i: (0, 0)),
            pl.BlockSpec((1, D), lambda i: (0, 0)),
            pl.BlockSpec((D, DFF), lambda i: (0, 0)),
            pl.BlockSpec((1, DFF), lambda i: (0, 0)),
            pl.BlockSpec((DFF, D), lambda i: (0, 0)),
            pl.BlockSpec((1, D), lambda i: (0, 0)),
        ],
        out_specs=pl.BlockSpec((blk, D), lambda i: (i, 0)),
        out_shape=jax.ShapeDtypeStruct((S, D), jnp.float32),
    )(stdh, mem, res, pw, pb.reshape(1, D), g_val.reshape(1, 1),
      g2.reshape(1, D), b2.reshape(1, D), w1, b1.reshape(1, DFF),
      w2, bb2.reshape(1, D))


def kernel(previous_hidden, attention_mask, head_mask, g_val, ln1_g, ln1_b,
           c_attn_w, c_attn_b, c_proj_w, c_proj_b, ln2_g, ln2_b,
           mlp_fc_w, mlp_fc_b, mlp_proj_w, mlp_proj_b, db_kv):
    x = previous_hidden.reshape(S, D)
    qkv = _ln_qkv(x, ln1_g, ln1_b, c_attn_w, c_attn_b)
    q = jax.lax.slice(qkv, (0, 0), (S, D))
    k = jax.lax.slice(qkv, (0, D), (S, 2 * D))
    v = jax.lax.slice(qkv, (0, 2 * D), (S, 3 * D))

    scores, cm_t = _scores(q, db_kv)
    flat = _topchunks(cm_t)
    cand = _sc_gather(scores.reshape(S * NCHUNK, CH), flat.reshape(-1),
                      chunk=128)
    idx = _topcand(cand.reshape(K, S, CH), flat)
    mem_kv = _sc_gather(db_kv, idx.reshape(-1),
                        chunk=32).reshape(S, K, 2, D)

    mem_merged = _memattn(q, mem_kv)
    stdh = _causal_attn(q, k, v, attention_mask, head_mask)
    out = _tail(stdh, mem_merged, x, c_proj_w, c_proj_b, g_val,
                ln2_g, ln2_b, mlp_fc_w, mlp_fc_b, mlp_proj_w, mlp_proj_b)
    return out.reshape(B, S, D)


# speedup vs baseline: 1.0785x; 1.0785x over previous
import functools

import jax
import jax.numpy as jnp
from jax import lax
from jax.experimental import pallas as pl
from jax.experimental.pallas import tpu as pltpu
from jax.experimental.pallas import tpu_sc as plsc

B, S, D = 1, 2048, 768
NH, DH = 12, 64
M = 32768
K = 32
DFF = 3072

CH = 128
NCHUNK = M // CH

NEG_INF = float(jnp.finfo(jnp.float32).min)
BIG_F = float(jnp.finfo(jnp.float32).max)

NC, NS = 2, 16
NW = NC * NS


def _ln_qkv_body(x_ref, g_ref, b_ref, w_ref, wb_ref, qkv_ref):
    x = x_ref[...]
    mu = jnp.mean(x, axis=-1, keepdims=True)
    var = jnp.mean((x - mu) ** 2, axis=-1, keepdims=True)
    h = (x - mu) * jax.lax.rsqrt(var + 1e-5) * g_ref[...] + b_ref[...]
    qkv_ref[...] = (
        jnp.dot(h, w_ref[...], preferred_element_type=jnp.float32) + wb_ref[...]
    )


def _ln_qkv(x, g, b, w, wb):
    blk = 256
    return pl.pallas_call(
        _ln_qkv_body,
        grid=(S // blk,),
        in_specs=[
            pl.BlockSpec((blk, D), lambda i: (i, 0)),
            pl.BlockSpec((1, D), lambda i: (0, 0)),
            pl.BlockSpec((1, D), lambda i: (0, 0)),
            pl.BlockSpec((D, 3 * D), lambda i: (0, 0)),
            pl.BlockSpec((1, 3 * D), lambda i: (0, 0)),
        ],
        out_specs=pl.BlockSpec((blk, 3 * D), lambda i: (i, 0)),
        out_shape=jax.ShapeDtypeStruct((S, 3 * D), jnp.float32),
    )(x, g.reshape(1, D), b.reshape(1, D), w, wb.reshape(1, 3 * D))


def _scores_body(q_ref, k_ref, s_ref, cm_ref, *, sblk, mblk):
    q = q_ref[...]
    k = k_ref[...]
    s = jax.lax.dot_general(
        q, k, (((1,), (1,)), ((), ())), preferred_element_type=jnp.float32
    )
    s_ref[...] = s
    cm = jnp.max(s.reshape(sblk, mblk // CH, CH), axis=-1)
    cm_ref[...] = cm.T


def _scores(q, db_flat):
    sblk, mblk = 256, 4096
    return pl.pallas_call(
        functools.partial(_scores_body, sblk=sblk, mblk=mblk),
        grid=(M // mblk, S // sblk),
        in_specs=[
            pl.BlockSpec((sblk, D), lambda m, s: (s, 0)),
            pl.BlockSpec((mblk, D), lambda m, s: (m, 0)),
        ],
        out_specs=[
            pl.BlockSpec((sblk, mblk), lambda m, s: (s, m)),
            pl.BlockSpec((mblk // CH, sblk), lambda m, s: (m, s)),
        ],
        out_shape=[
            jax.ShapeDtypeStruct((S, M), jnp.float32),
            jax.ShapeDtypeStruct((NCHUNK, S), jnp.float32),
        ],
    )(q, db_flat)


def _t1_body(cm_ref, o_ref):
    cm = cm_ref[...]
    iota = jax.lax.broadcasted_iota(jnp.int32, (NCHUNK, S), 0)
    lane = jax.lax.broadcasted_iota(jnp.int32, (1, S), 1)
    big = jnp.int32(2**30)
    rows = []
    for _ in range(K):
        m = jnp.max(cm, axis=0, keepdims=True)
        pos = jnp.where(cm == m, iota, big)
        amin = jnp.min(pos, axis=0, keepdims=True)
        rows.append(amin + lane * NCHUNK)
        cm = jnp.where(iota == amin, NEG_INF, cm)
    o_ref[...] = jnp.concatenate(rows, axis=0)


def _topchunks(cm_t):
    return pl.pallas_call(
        _t1_body,
        grid=(1,),
        in_specs=[pl.BlockSpec((NCHUNK, S), lambda i: (0, 0))],
        out_specs=pl.BlockSpec((K, S), lambda i: (0, 0)),
        out_shape=jax.ShapeDtypeStruct((K, S), jnp.int32),
    )(cm_t)


def _sc_gather(table, idx, chunk):
    row_shape = table.shape[1:]
    N = idx.shape[0]
    n_per_w = N // NW
    nch = n_per_w // chunk
    mesh = plsc.VectorSubcoreMesh(
        core_axis_name="c", subcore_axis_name="s", num_cores=NC,
        num_subcores=NS)

    npair = nch // 2

    @functools.partial(
        pl.kernel, mesh=mesh,
        out_type=jax.ShapeDtypeStruct((N,) + row_shape, jnp.float32),
        scratch_types=[
            pltpu.VMEM((chunk,), jnp.int32),
            pltpu.VMEM((chunk,), jnp.int32),
            pltpu.VMEM((chunk,) + row_shape, jnp.float32),
            pltpu.VMEM((chunk,) + row_shape, jnp.float32),
            pltpu.SemaphoreType.DMA,
            pltpu.SemaphoreType.DMA,
        ],
    )
    def k(table_hbm, idx_hbm, out_hbm, ic0, ic1, b0, b1, sem0, sem1):
        wid = lax.axis_index("s") * NC + lax.axis_index("c")
        base = wid * n_per_w

        def start(icr, bufr, semr, c):
            pltpu.sync_copy(idx_hbm.at[pl.ds(base + c * chunk, chunk)], icr)
            pltpu.async_copy(table_hbm.at[icr], bufr, semr)

        def drain(icr, bufr, semr, c):
            pltpu.make_async_copy(table_hbm.at[icr], bufr, semr).wait()
            pltpu.sync_copy(bufr, out_hbm.at[pl.ds(base + c * chunk, chunk)])

        start(ic0, b0, sem0, 0)

        def body(p, carry):
            start(ic1, b1, sem1, 2 * p + 1)
            drain(ic0, b0, sem0, 2 * p)

            @pl.when(p < npair - 1)
            def _():
                start(ic0, b0, sem0, 2 * p + 2)

            drain(ic1, b1, sem1, 2 * p + 1)
            return carry

        lax.fori_loop(0, npair, body, 0)

    return k(table, idx)


def _t2_body(cand_ref, flat_ref, o_ref, *, rblk):
    qb = pl.program_id(0)
    cand = cand_ref[...]
    lane = jax.lax.broadcasted_iota(jnp.int32, (1, rblk, CH), 2)
    l_io = jax.lax.broadcasted_iota(jnp.int32, (K, rblk, 1), 1) + qb * rblk
    cid = flat_ref[...].reshape(K, rblk, 1) - l_io * NCHUNK
    g3 = cid * CH + lane
    cols = []
    for _ in range(K):
        m1 = jnp.max(cand, axis=2, keepdims=True)
        m = jnp.max(m1, axis=0, keepdims=True)
        hit = cand == m
        s1 = jnp.min(jnp.where(hit, g3, jnp.int32(2**30)), axis=2,
                     keepdims=True)
        sel = jnp.min(s1, axis=0, keepdims=True)
        cols.append(sel.reshape(rblk, 1))
        cand = jnp.where(g3 == sel, NEG_INF, cand)
    o_ref[...] = jnp.concatenate(cols, axis=-1)


def _topcand(cand3, flat):
    rblk = 256
    return pl.pallas_call(
        functools.partial(_t2_body, rblk=rblk),
        grid=(S // rblk,),
        in_specs=[
            pl.BlockSpec((K, rblk, CH), lambda i: (0, i, 0)),
            pl.BlockSpec((K, rblk), lambda i: (0, i)),
        ],
        out_specs=pl.BlockSpec((rblk, K), lambda i: (i, 0)),
        out_shape=jax.ShapeDtypeStruct((S, K), jnp.int32),
    )(cand3, flat)


def _memattn_body(q_ref, kv_ref, o_ref):
    q = q_ref[...]
    outs = []
    scale = 1.0 / jnp.sqrt(jnp.float32(DH))
    for h in range(NH):
        qh = q[:, h * DH:(h + 1) * DH]
        mkh = kv_ref[:, :, h * DH:(h + 1) * DH]
        mvh = kv_ref[:, :, D + h * DH:D + (h + 1) * DH]
        aw = jnp.sum(qh[:, None, :] * mkh, axis=-1) * scale
        aw = aw - jnp.max(aw, axis=-1, keepdims=True)
        aw = jnp.exp(aw)
        aw = aw / jnp.sum(aw, axis=-1, keepdims=True)
        outs.append(jnp.sum(aw[:, :, None] * mvh, axis=1))
    o_ref[...] = jnp.concatenate(outs, axis=-1)


def _memattn(q, mem_kv_flat):
    blk = 64
    return pl.pallas_call(
        _memattn_body,
        grid=(S // blk,),
        in_specs=[
            pl.BlockSpec((blk, D), lambda i: (i, 0)),
            pl.BlockSpec((blk, K, 2 * D), lambda i: (i, 0, 0)),
        ],
        out_specs=pl.BlockSpec((blk, D), lambda i: (i, 0)),
        out_shape=jax.ShapeDtypeStruct((S, D), jnp.float32),
    )(q, mem_kv_flat)


def _causal_body(q_ref, k_ref, v_ref, am_ref, hm_ref, o_ref,
                 acc_ref, m_ref, l_ref, *, qblk, kblk):
    qb = pl.program_id(0)
    kb = pl.program_id(1)
    nkb = pl.num_programs(1)
    scale = 1.0 / jnp.sqrt(jnp.float32(DH))

    @pl.when(kb == 0)
    def _init():
        acc_ref[...] = jnp.zeros_like(acc_ref)
        m_ref[...] = jnp.full_like(m_ref, NEG_INF)
        l_ref[...] = jnp.zeros_like(l_ref)

    @pl.when(kb <= qb)
    def _compute():
        rows = jax.lax.broadcasted_iota(jnp.int32, (qblk, kblk), 0) + qb * qblk
        cols = jax.lax.broadcasted_iota(jnp.int32, (qblk, kblk), 1) + kb * kblk
        causal = rows >= cols
        am = am_ref[...]
        for h in range(NH):
            sl = slice(h * DH, (h + 1) * DH)
            logits = jax.lax.dot_general(
                q_ref[:, sl], k_ref[:, sl], (((1,), (1,)), ((), ())),
                preferred_element_type=jnp.float32) * scale
            logits = jnp.where(causal, logits, NEG_INF) + am
            mo = m_ref[:, h:h + 1]
            mn = jnp.maximum(mo, jnp.max(logits, axis=-1, keepdims=True))
            p = jnp.exp(logits - mn)
            corr = jnp.exp(mo - mn)
            l_ref[:, h:h + 1] = l_ref[:, h:h + 1] * corr + jnp.sum(
                p, axis=-1, keepdims=True)
            acc_ref[:, sl] = acc_ref[:, sl] * corr + jnp.dot(
                p, v_ref[:, sl], preferred_element_type=jnp.float32)
            m_ref[:, h:h + 1] = mn

    @pl.when(kb == nkb - 1)
    def _final():
        outs = []
        for h in range(NH):
            sl = slice(h * DH, (h + 1) * DH)
            outs.append(acc_ref[:, sl] / l_ref[:, h:h + 1] * hm_ref[0, h])
        o_ref[...] = jnp.concatenate(outs, axis=-1)


def _causal_attn(q, k, v, amask, hmask):
    qblk, kblk = 256, 256
    return pl.pallas_call(
        functools.partial(_causal_body, qblk=qblk, kblk=kblk),
        grid=(S // qblk, S // kblk),
        in_specs=[
            pl.BlockSpec((qblk, D), lambda i, j: (i, 0)),
            pl.BlockSpec((kblk, D), lambda i, j: (j, 0)),
            pl.BlockSpec((kblk, D), lambda i, j: (j, 0)),
            pl.BlockSpec((1, kblk), lambda i, j: (0, j)),
            pl.BlockSpec((1, NH), lambda i, j: (0, 0)),
        ],
        out_specs=pl.BlockSpec((qblk, D), lambda i, j: (i, 0)),
        out_shape=jax.ShapeDtypeStruct((S, D), jnp.float32),
        scratch_shapes=[
            pltpu.VMEM((qblk, D), jnp.float32),
            pltpu.VMEM((qblk, NH), jnp.float32),
            pltpu.VMEM((qblk, NH), jnp.float32),
        ],
    )(q, k, v, amask.reshape(1, S), hmask.reshape(1, NH))


def _tail_body(stdh_ref, mem_ref, res_ref, pw_ref, pb_ref, g_ref,
               g2_ref, b2_ref, w1_ref, b1_ref, w2_ref, bb2_ref, o_ref):
    std = (
        jnp.dot(stdh_ref[...], pw_ref[...], preferred_element_type=jnp.float32)
        + pb_ref[...]
    )
    g = g_ref[0, 0]
    attn = (1.0 - g) * std + g * mem_ref[...]
    hidden = attn + res_ref[...]
    mu = jnp.mean(hidden, axis=-1, keepdims=True)
    var = jnp.mean((hidden - mu) ** 2, axis=-1, keepdims=True)
    h2 = (hidden - mu) * jax.lax.rsqrt(var + 1e-5) * g2_ref[...] + b2_ref[...]
    ff = jnp.dot(h2, w1_ref[...], preferred_element_type=jnp.float32) + b1_ref[...]
    ff = jax.nn.gelu(ff, approximate=True)
    ff = jnp.dot(ff, w2_ref[...], preferred_element_type=jnp.float32) + bb2_ref[...]
    o_ref[...] = hidden + ff


def _tail(stdh, mem, res, pw, pb, g_val, g2, b2, w1, b1, w2, bb2):
    blk = 256
    return pl.pallas_call(
        _tail_body,
        grid=(S // blk,),
        in_specs=[
            pl.BlockSpec((blk, D), lambda i: (i, 0)),
            pl.BlockSpec((blk, D), lambda i: (i, 0)),
            pl.BlockSpec((blk, D), lambda i: (i, 0)),
            pl.BlockSpec((D, D), lambda i: (0, 0)),
            pl.BlockSpec((1, D), lambda i: (0, 0)),
            pl.BlockSpec((1, 1), lambda i: (0, 0)),
            pl.BlockSpec((1, D), lambda i: (0, 0)),
            pl.BlockSpec((1, D), lambda i: (0, 0)),
            pl.BlockSpec((D, DFF), lambda i: (0, 0)),
            pl.BlockSpec((1, DFF), lambda i: (0, 0)),
            pl.BlockSpec((DFF, D), lambda i: (0, 0)),
            pl.BlockSpec((1, D), lambda i: (0, 0)),
        ],
        out_specs=pl.BlockSpec((blk, D), lambda i: (i, 0)),
        out_shape=jax.ShapeDtypeStruct((S, D), jnp.float32),
    )(stdh, mem, res, pw, pb.reshape(1, D), g_val.reshape(1, 1),
      g2.reshape(1, D), b2.reshape(1, D), w1, b1.reshape(1, DFF),
      w2, bb2.reshape(1, D))


def kernel(previous_hidden, attention_mask, head_mask, g_val, ln1_g, ln1_b,
           c_attn_w, c_attn_b, c_proj_w, c_proj_b, ln2_g, ln2_b,
           mlp_fc_w, mlp_fc_b, mlp_proj_w, mlp_proj_b, db_kv):
    x = previous_hidden.reshape(S, D)
    qkv = _ln_qkv(x, ln1_g, ln1_b, c_attn_w, c_attn_b)
    q = jax.lax.slice(qkv, (0, 0), (S, D))
    k = jax.lax.slice(qkv, (0, D), (S, 2 * D))
    v = jax.lax.slice(qkv, (0, 2 * D), (S, 3 * D))

    db_flat = db_kv.reshape(M, 2 * D)
    scores, cm_t = _scores(q, db_flat)
    flat = _topchunks(cm_t)
    cand = _sc_gather(scores.reshape(S * NCHUNK, CH), flat.reshape(-1),
                      chunk=128)
    idx = _topcand(cand.reshape(K, S, CH), flat)
    mem_kv = _sc_gather(db_flat, idx.reshape(-1),
                        chunk=32).reshape(S, K, 2 * D)

    mem_merged = _memattn(q, mem_kv)
    stdh = _causal_attn(q, k, v, attention_mask, head_mask)
    out = _tail(stdh, mem_merged, x, c_proj_w, c_proj_b, g_val,
                ln2_g, ln2_b, mlp_fc_w, mlp_fc_b, mlp_proj_w, mlp_proj_b)
    return out.reshape(B, S, D)


# speedup vs baseline: 1.1010x; 1.0209x over previous
import functools

import jax
import jax.numpy as jnp
from jax import lax
from jax.experimental import pallas as pl
from jax.experimental.pallas import tpu as pltpu
from jax.experimental.pallas import tpu_sc as plsc

B, S, D = 1, 2048, 768
NH, DH = 12, 64
M = 32768
K = 32
DFF = 3072

CH = 128
NCHUNK = M // CH

NEG_INF = float(jnp.finfo(jnp.float32).min)
BIG_F = float(jnp.finfo(jnp.float32).max)

NC, NS = 2, 16
NW = NC * NS


def _ln_qkv_body(x_ref, g_ref, b_ref, w_ref, wb_ref, qkv_ref):
    x = x_ref[...]
    mu = jnp.mean(x, axis=-1, keepdims=True)
    var = jnp.mean((x - mu) ** 2, axis=-1, keepdims=True)
    h = (x - mu) * jax.lax.rsqrt(var + 1e-5) * g_ref[...] + b_ref[...]
    qkv_ref[...] = (
        jnp.dot(h, w_ref[...], preferred_element_type=jnp.float32) + wb_ref[...]
    )


def _ln_qkv(x, g, b, w, wb):
    blk = 256
    return pl.pallas_call(
        _ln_qkv_body,
        grid=(S // blk,),
        in_specs=[
            pl.BlockSpec((blk, D), lambda i: (i, 0)),
            pl.BlockSpec((1, D), lambda i: (0, 0)),
            pl.BlockSpec((1, D), lambda i: (0, 0)),
            pl.BlockSpec((D, 3 * D), lambda i: (0, 0)),
            pl.BlockSpec((1, 3 * D), lambda i: (0, 0)),
        ],
        out_specs=pl.BlockSpec((blk, 3 * D), lambda i: (i, 0)),
        out_shape=jax.ShapeDtypeStruct((S, 3 * D), jnp.float32),
    )(x, g.reshape(1, D), b.reshape(1, D), w, wb.reshape(1, 3 * D))


def _scores_body(q_ref, k_ref, s_ref, cm_ref, *, sblk, mblk):
    q = q_ref[...]
    k = k_ref[...]
    s = jax.lax.dot_general(
        q, k, (((1,), (1,)), ((), ())), preferred_element_type=jnp.float32
    )
    s_ref[...] = s
    cm = jnp.max(s.reshape(sblk, mblk // CH, CH), axis=-1)
    cm_ref[...] = cm.T


def _scores(q, db_flat):
    sblk, mblk = 256, 4096
    return pl.pallas_call(
        functools.partial(_scores_body, sblk=sblk, mblk=mblk),
        grid=(M // mblk, S // sblk),
        in_specs=[
            pl.BlockSpec((sblk, D), lambda m, s: (s, 0)),
            pl.BlockSpec((mblk, D), lambda m, s: (m, 0)),
        ],
        out_specs=[
            pl.BlockSpec((sblk, mblk), lambda m, s: (s, m)),
            pl.BlockSpec((mblk // CH, sblk), lambda m, s: (m, s)),
        ],
        out_shape=[
            jax.ShapeDtypeStruct((S, M), jnp.float32),
            jax.ShapeDtypeStruct((NCHUNK, S), jnp.float32),
        ],
    )(q, db_flat)


def _t1_body(cm_ref, o_ref):
    cm = cm_ref[...]
    iota = jax.lax.broadcasted_iota(jnp.int32, (NCHUNK, S), 0)
    lane = jax.lax.broadcasted_iota(jnp.int32, (1, S), 1)
    big = jnp.int32(2**30)
    rows = []
    for _ in range(K):
        m = jnp.max(cm, axis=0, keepdims=True)
        pos = jnp.where(cm == m, iota, big)
        amin = jnp.min(pos, axis=0, keepdims=True)
        rows.append(amin + lane * NCHUNK)
        cm = jnp.where(iota == amin, NEG_INF, cm)
    o_ref[...] = jnp.concatenate(rows, axis=0)


def _topchunks(cm_t):
    return pl.pallas_call(
        _t1_body,
        grid=(1,),
        in_specs=[pl.BlockSpec((NCHUNK, S), lambda i: (0, 0))],
        out_specs=pl.BlockSpec((K, S), lambda i: (0, 0)),
        out_shape=jax.ShapeDtypeStruct((K, S), jnp.int32),
    )(cm_t)


def _sc_gather(table, idx, chunk):
    row_shape = table.shape[1:]
    N = idx.shape[0]
    n_per_w = N // NW
    nch = n_per_w // chunk
    mesh = plsc.VectorSubcoreMesh(
        core_axis_name="c", subcore_axis_name="s", num_cores=NC,
        num_subcores=NS)

    npair = nch // 2

    @functools.partial(
        pl.kernel, mesh=mesh,
        out_type=jax.ShapeDtypeStruct((N,) + row_shape, jnp.float32),
        scratch_types=[
            pltpu.VMEM((chunk,), jnp.int32),
            pltpu.VMEM((chunk,), jnp.int32),
            pltpu.VMEM((chunk,) + row_shape, jnp.float32),
            pltpu.VMEM((chunk,) + row_shape, jnp.float32),
            pltpu.SemaphoreType.DMA,
            pltpu.SemaphoreType.DMA,
        ],
    )
    def k(table_hbm, idx_hbm, out_hbm, ic0, ic1, b0, b1, sem0, sem1):
        wid = lax.axis_index("s") * NC + lax.axis_index("c")
        base = wid * n_per_w

        def start(icr, bufr, semr, c):
            pltpu.sync_copy(idx_hbm.at[pl.ds(base + c * chunk, chunk)], icr)
            pltpu.async_copy(table_hbm.at[icr], bufr, semr)

        def drain(icr, bufr, semr, c):
            pltpu.make_async_copy(table_hbm.at[icr], bufr, semr).wait()
            pltpu.sync_copy(bufr, out_hbm.at[pl.ds(base + c * chunk, chunk)])

        start(ic0, b0, sem0, 0)

        def body(p, carry):
            start(ic1, b1, sem1, 2 * p + 1)
            drain(ic0, b0, sem0, 2 * p)

            @pl.when(p < npair - 1)
            def _():
                start(ic0, b0, sem0, 2 * p + 2)

            drain(ic1, b1, sem1, 2 * p + 1)
            return carry

        lax.fori_loop(0, npair, body, 0)

    return k(table, idx)


def _t2_body(cand_ref, flat_ref, o_ref, *, rblk):
    qb = pl.program_id(0)
    cand = cand_ref[...]
    lane = jax.lax.broadcasted_iota(jnp.int32, (1, rblk, CH), 2)
    l_io = jax.lax.broadcasted_iota(jnp.int32, (K, rblk, 1), 1) + qb * rblk
    cid = flat_ref[...].reshape(K, rblk, 1) - l_io * NCHUNK
    g3 = cid * CH + lane
    cols = []
    for _ in range(K):
        m1 = jnp.max(cand, axis=2, keepdims=True)
        m = jnp.max(m1, axis=0, keepdims=True)
        hit = cand == m
        s1 = jnp.min(jnp.where(hit, g3, jnp.int32(2**30)), axis=2,
                     keepdims=True)
        sel = jnp.min(s1, axis=0, keepdims=True)
        cols.append(sel.reshape(rblk, 1))
        cand = jnp.where(g3 == sel, NEG_INF, cand)
    o_ref[...] = jnp.concatenate(cols, axis=-1)


def _topcand(cand3, flat):
    rblk = 256
    return pl.pallas_call(
        functools.partial(_t2_body, rblk=rblk),
        grid=(S // rblk,),
        in_specs=[
            pl.BlockSpec((K, rblk, CH), lambda i: (0, i, 0)),
            pl.BlockSpec((K, rblk), lambda i: (0, i)),
        ],
        out_specs=pl.BlockSpec((rblk, K), lambda i: (i, 0)),
        out_shape=jax.ShapeDtypeStruct((S, K), jnp.int32),
    )(cand3, flat)


def _memattn_body(q_ref, kv_ref, o_ref):
    q = q_ref[...]
    outs = []
    scale = 1.0 / jnp.sqrt(jnp.float32(DH))
    for h in range(NH):
        qh = q[:, h * DH:(h + 1) * DH]
        mkh = kv_ref[:, :, h * DH:(h + 1) * DH]
        mvh = kv_ref[:, :, D + h * DH:D + (h + 1) * DH]
        aw = jnp.sum(qh[:, None, :] * mkh, axis=-1) * scale
        aw = aw - jnp.max(aw, axis=-1, keepdims=True)
        aw = jnp.exp(aw)
        aw = aw / jnp.sum(aw, axis=-1, keepdims=True)
        outs.append(jnp.sum(aw[:, :, None] * mvh, axis=1))
    o_ref[...] = jnp.concatenate(outs, axis=-1)


def _memattn(q, mem_kv_flat):
    blk = 64
    return pl.pallas_call(
        _memattn_body,
        grid=(S // blk,),
        in_specs=[
            pl.BlockSpec((blk, D), lambda i: (i, 0)),
            pl.BlockSpec((blk, K, 2 * D), lambda i: (i, 0, 0)),
        ],
        out_specs=pl.BlockSpec((blk, D), lambda i: (i, 0)),
        out_shape=jax.ShapeDtypeStruct((S, D), jnp.float32),
    )(q, mem_kv_flat)


def _causal_body(q_ref, k_ref, v_ref, am_ref, hm_ref, o_ref, *, qblk):
    qb = pl.program_id(0)
    rows = jax.lax.broadcasted_iota(jnp.int32, (qblk, S), 0) + qb * qblk
    cols = jax.lax.broadcasted_iota(jnp.int32, (qblk, S), 1)
    causal = rows >= cols
    am = am_ref[...]
    scale = 1.0 / jnp.sqrt(jnp.float32(DH))
    outs = []
    for h in range(NH):
        qh = q_ref[:, h * DH:(h + 1) * DH]
        kh = k_ref[:, h * DH:(h + 1) * DH]
        vh = v_ref[:, h * DH:(h + 1) * DH]
        logits = jax.lax.dot_general(
            qh, kh, (((1,), (1,)), ((), ())), preferred_element_type=jnp.float32
        ) * scale
        logits = jnp.where(causal, logits, NEG_INF) + am
        m = jnp.max(logits, axis=-1, keepdims=True)
        p = jnp.exp(logits - m)
        p = p / jnp.sum(p, axis=-1, keepdims=True)
        p = p * hm_ref[0, h]
        outs.append(jnp.dot(p, vh, preferred_element_type=jnp.float32))
    o_ref[...] = jnp.concatenate(outs, axis=-1)


def _causal_attn(q, k, v, amask, hmask):
    qblk = 256
    return pl.pallas_call(
        functools.partial(_causal_body, qblk=qblk),
        grid=(S // qblk,),
        in_specs=[
            pl.BlockSpec((qblk, D), lambda i: (i, 0)),
            pl.BlockSpec((S, D), lambda i: (0, 0)),
            pl.BlockSpec((S, D), lambda i: (0, 0)),
            pl.BlockSpec((1, S), lambda i: (0, 0)),
            pl.BlockSpec((1, NH), lambda i: (0, 0)),
        ],
        out_specs=pl.BlockSpec((qblk, D), lambda i: (i, 0)),
        out_shape=jax.ShapeDtypeStruct((S, D), jnp.float32),
    )(q, k, v, amask.reshape(1, S), hmask.reshape(1, NH))


def _tail_body(stdh_ref, mem_ref, res_ref, pw_ref, pb_ref, g_ref,
               g2_ref, b2_ref, w1_ref, b1_ref, w2_ref, bb2_ref, o_ref):
    std = (
        jnp.dot(stdh_ref[...], pw_ref[...], preferred_element_type=jnp.float32)
        + pb_ref[...]
    )
    g = g_ref[0, 0]
    attn = (1.0 - g) * std + g * mem_ref[...]
    hidden = attn + res_ref[...]
    mu = jnp.mean(hidden, axis=-1, keepdims=True)
    var = jnp.mean((hidden - mu) ** 2, axis=-1, keepdims=True)
    h2 = (hidden - mu) * jax.lax.rsqrt(var + 1e-5) * g2_ref[...] + b2_ref[...]
    ff = jnp.dot(h2, w1_ref[...], preferred_element_type=jnp.float32) + b1_ref[...]
    ff = jax.nn.gelu(ff, approximate=True)
    ff = jnp.dot(ff, w2_ref[...], preferred_element_type=jnp.float32) + bb2_ref[...]
    o_ref[...] = hidden + ff


def _tail(stdh, mem, res, pw, pb, g_val, g2, b2, w1, b1, w2, bb2):
    blk = 256
    return pl.pallas_call(
        _tail_body,
        grid=(S // blk,),
        in_specs=[
            pl.BlockSpec((blk, D), lambda i: (i, 0)),
            pl.BlockSpec((blk, D), lambda i: (i, 0)),
            pl.BlockSpec((blk, D), lambda i: (i, 0)),
            pl.BlockSpec((D, D), lambda i: (0, 0)),
            pl.BlockSpec((1, D), lambda i: (0, 0)),
            pl.BlockSpec((1, 1), lambda i: (0, 0)),
            pl.BlockSpec((1, D), lambda i: (0, 0)),
            pl.BlockSpec((1, D), lambda i: (0, 0)),
            pl.BlockSpec((D, DFF), lambda i: (0, 0)),
            pl.BlockSpec((1, DFF), lambda i: (0, 0)),
            pl.BlockSpec((DFF, D), lambda i: (0, 0)),
            pl.BlockSpec((1, D), lambda i: (0, 0)),
        ],
        out_specs=pl.BlockSpec((blk, D), lambda i: (i, 0)),
        out_shape=jax.ShapeDtypeStruct((S, D), jnp.float32),
    )(stdh, mem, res, pw, pb.reshape(1, D), g_val.reshape(1, 1),
      g2.reshape(1, D), b2.reshape(1, D), w1, b1.reshape(1, DFF),
      w2, bb2.reshape(1, D))


def kernel(previous_hidden, attention_mask, head_mask, g_val, ln1_g, ln1_b,
           c_attn_w, c_attn_b, c_proj_w, c_proj_b, ln2_g, ln2_b,
           mlp_fc_w, mlp_fc_b, mlp_proj_w, mlp_proj_b, db_kv):
    x = previous_hidden.reshape(S, D)
    qkv = _ln_qkv(x, ln1_g, ln1_b, c_attn_w, c_attn_b)
    q = jax.lax.slice(qkv, (0, 0), (S, D))
    k = jax.lax.slice(qkv, (0, D), (S, 2 * D))
    v = jax.lax.slice(qkv, (0, 2 * D), (S, 3 * D))

    db_flat = db_kv.reshape(M, 2 * D)
    scores, cm_t = _scores(q, db_flat)
    flat = _topchunks(cm_t)
    cand = _sc_gather(scores.reshape(S * NCHUNK, CH), flat.reshape(-1),
                      chunk=128)
    idx = _topcand(cand.reshape(K, S, CH), flat)
    mem_kv = _sc_gather(db_flat, idx.reshape(-1),
                        chunk=32).reshape(S, K, 2 * D)

    mem_merged = _memattn(q, mem_kv)
    stdh = _causal_attn(q, k, v, attention_mask, head_mask)
    out = _tail(stdh, mem_merged, x, c_proj_w, c_proj_b, g_val,
                ln2_g, ln2_b, mlp_fc_w, mlp_fc_b, mlp_proj_w, mlp_proj_b)
    return out.reshape(B, S, D)
